# baseline (speedup 1.0000x reference)
import functools

import jax
import jax.numpy as jnp
from jax import lax
from jax.experimental import pallas as pl
from jax.experimental.pallas import tpu as pltpu
from jax.experimental.pallas import tpu_sc as plsc

_NG = 16
_F32 = jnp.float32

_dot = functools.partial(
    lax.dot, precision=lax.Precision.HIGHEST, preferred_element_type=_F32)


def _dott(a, b):
    return lax.dot_general(
        a, b, (((0,), (0,)), ((), ())),
        precision=lax.Precision.HIGHEST, preferred_element_type=_F32)


def _ln(x, g, b):
    m = jnp.mean(x, axis=-1, keepdims=True)
    v = jnp.mean((x - m) ** 2, axis=-1, keepdims=True)
    return (x - m) * lax.rsqrt(v + 1e-5) * g + b


def _edge_hist_sc(src2d, dstp, batch, n_rows):
    N = batch.shape[0]
    EP = dstp.shape[0]
    NW = 32
    EPT = EP // NW
    NDMA = EPT // 128
    GROUPS = EPT // 16
    RPT = n_rows // 16
    mesh = plsc.VectorSubcoreMesh(core_axis_name="c", subcore_axis_name="s")

    @functools.partial(
        pl.kernel,
        mesh=mesh,
        compiler_params=pltpu.CompilerParams(
            needs_layout_passes=False, use_tc_tiling_on_sc=False),
        out_type=jax.ShapeDtypeStruct((2, n_rows, _NG), _F32),
        scratch_types=[
            pltpu.VMEM((NDMA, 128), jnp.int32),
            pltpu.VMEM((EPT,), jnp.int32),
            pltpu.VMEM((N,), jnp.int32),
            pltpu.VMEM((EPT, _NG), _F32),
            pltpu.VMEM_SHARED((n_rows, _NG), _F32),
            pltpu.SemaphoreType.DMA,
        ],
    )
    def k(src_hbm, dst_hbm, batch_hbm, out_hbm, srcv, dstv, bv, oh, csh, sem):
        c = lax.axis_index("c")
        s = lax.axis_index("s")
        tile = c * 16 + s
        pltpu.sync_copy(src_hbm.at[pl.ds(tile * NDMA, NDMA)], srcv)
        pltpu.sync_copy(dst_hbm.at[pl.ds(tile * EPT, EPT)], dstv)
        pltpu.sync_copy(batch_hbm, bv)

        zero16 = jnp.zeros((16,), _F32)

        def zero_body(r, _):
            oh[r, :] = zero16
            return 0

        lax.fori_loop(0, EPT, zero_body, 0)
        pltpu.sync_copy(oh.at[pl.ds(0, RPT)], csh.at[pl.ds(s * RPT, RPT)])
        plsc.subcore_barrier()

        ones16 = jnp.ones((16,), _F32)
        iota16 = lax.iota(jnp.int32, 16)

        def fill_body(g, _):
            dvec = dstv[pl.ds(g * 16, 16)]
            bg = plsc.load_gather(bv, [dvec])
            rows = g * 16 + iota16
            plsc.store_scatter(oh, [rows, bg], ones16)
            return 0

        lax.fori_loop(0, GROUPS, fill_body, 0)

        def fire_body(j, _):
            pltpu.async_copy(
                oh.at[pl.ds(j * 128, 128)], csh.at[srcv.at[j]], sem, add=True)
            return 0

        lax.fori_loop(0, NDMA, fire_body, 0)

        def drain_body(j, _):
            pltpu.make_async_copy(
                oh.at[pl.ds(j * 128, 128)], csh.at[srcv.at[j]], sem).wait()
            return 0

        lax.fori_loop(0, NDMA, drain_body, 0)
        plsc.subcore_barrier()

        pltpu.sync_copy(csh.at[pl.ds(s * RPT, RPT)], oh.at[pl.ds(0, RPT)])
        pltpu.sync_copy(oh.at[pl.ds(0, RPT)], out_hbm.at[c, pl.ds(s * RPT, RPT)])

    return k(src2d, dstp, batch)


def _enc_body(x_r, b_r, lpe_r, rw_r, W1, b1, g1, bn1, W2, b2,
              thh, thl, thr, h_out, axh, axl, axr, m_out, cnt_out):
    i = pl.program_id(0)
    t = _dot(x_r[...], W1[...]) + b1[...]
    t = jnp.maximum(_ln(t, g1[...], bn1[...]), 0.0)
    h = _dot(t, W2[...]) + b2[...]
    h_out[...] = h
    lpe = lpe_r[...]
    rw = rw_r[...]
    alpha64 = _dot(h, thh[...]) + _dot(lpe, thl[...]) + _dot(rw, thr[...])
    g64 = lax.broadcasted_iota(jnp.int32, (1, 4 * _NG), 1) // 4
    a2 = alpha64 * (b_r[...] == g64).astype(_F32)
    iota_g = lax.broadcasted_iota(jnp.int32, (1, _NG), 1)
    oh = (b_r[...] == iota_g).astype(_F32)

    @pl.when(i == 0)
    def _():
        axh[...] = jnp.zeros_like(axh)
        axl[...] = jnp.zeros_like(axl)
        axr[...] = jnp.zeros_like(axr)
        m_out[...] = jnp.zeros_like(m_out)
        cnt_out[...] = jnp.zeros_like(cnt_out)

    axh[...] += _dott(a2, h)
    axl[...] += _dott(a2, lpe)
    axr[...] += _dott(a2, rw)
    m_out[...] += _dott(oh, h)
    cnt_out[...] += _dott(oh, jnp.ones((h.shape[0], 1), _F32))


def _cred_body(h_r, b_r, c0_r, c1_r, mc_out, oc_out):
    i = pl.program_id(0)
    h = h_r[...]
    iota_g = lax.broadcasted_iota(jnp.int32, (1, _NG), 1)
    oh = (b_r[...] == iota_g).astype(_F32)
    cs = c0_r[...] + c1_r[...]

    @pl.when(i == 0)
    def _():
        mc_out[...] = jnp.zeros_like(mc_out)
        oc_out[...] = jnp.zeros_like(oc_out)

    mc_out[...] += _dott(cs, h)
    oc_out[...] += _dott(cs, oh)


def _mamba_body(u0, u1, u2, u3, m_r, mc_r, oc_r, cnt_r, inW, convWT, convb,
                xpW, dtW, dtb, Alog, Dp, outW, lng, lnb, mW1, mW2, mb, gW, gb,
                dW1, db1, dg, dbn, dW2, db2, out_r):
    us = [u0[...], u1[...], u2[...], u3[...]]
    di = convb.shape[-1]
    dr = dtW.shape[0]
    ds = Alog.shape[-1]
    xis = []
    z3 = None
    for t in range(4):
        xz = _dot(us[t], inW[...])
        xis.append(xz[:, :di])
        if t == 3:
            z3 = xz[:, di:]
    convs = []
    for t in range(4):
        acc = convb[...]
        for k in range(4):
            ti = t + k - 3
            if ti >= 0:
                acc = acc + xis[ti] * convWT[k, :][None, :]
        convs.append(acc * jax.nn.sigmoid(acc))
    dts, bms = [], []
    cm3 = None
    for t in range(4):
        dbc = _dot(convs[t], xpW[...])
        dts.append(jax.nn.softplus(_dot(dbc[:, :dr], dtW[...]) + dtb[...]))
        bms.append(dbc[:, dr:dr + ds])
        if t == 3:
            cm3 = dbc[:, dr + ds:dr + 2 * ds]
    A = -jnp.exp(Alog[...])
    h = jnp.zeros((us[0].shape[0], di, ds), _F32)
    for t in range(4):
        dA = jnp.exp(dts[t][:, :, None] * A[None, :, :])
        h = dA * h + (dts[t] * convs[t])[:, :, None] * bms[t][:, None, :]
    y3 = jnp.sum(h * cm3[:, None, :], axis=-1) + Dp[...] * convs[3]
    y3 = y3 * (z3 * jax.nn.sigmoid(z3))
    xm = _ln(_dot(y3, outW[...]), lng[...], lnb[...])
    q = _dot(xm, mW2[...]) + mb[...]

    cnt = cnt_r[...]
    iota_r = lax.broadcasted_iota(jnp.int32, (_NG, _NG), 0)
    iota_c = lax.broadcasted_iota(jnp.int32, (_NG, _NG), 1)
    o_tot = jnp.where(iota_r == iota_c, cnt, 0.0) + oc_r[...]
    s_tot = _dot(m_r[...] + mc_r[...], mW1[...]) + _dot(o_tot, q)
    pooled = _dot(s_tot, gW[...]) + cnt_r[...] * gb[...]
    t2 = _dot(pooled, dW1[...]) + db1[...]
    t2 = jnp.maximum(_ln(t2, dg[...], dbn[...]), 0.0)
    out_r[...] = _dot(t2, dW2[...]) + db2[...]


def kernel(x, edge_index, batch, laplacePE, rwse, params):
    p = params
    N, D = x.shape
    E = edge_index.shape[1]
    H = p['enc_W2'].shape[1]
    PE = laplacePE.shape[1] + rwse.shape[1]
    dm = H + PE
    OUT = p['dec_W2'].shape[1]
    BN = 2000
    NB = N // BN

    batch_i32 = batch.astype(jnp.int32)
    b2d = batch_i32.reshape(N, 1)

    EPT = 5120
    EP = 32 * EPT
    NR = ((N + 16 + 127) // 128) * 128
    src = edge_index[0].astype(jnp.int32)
    dst = edge_index[1].astype(jnp.int32)
    pad_src = N + (jnp.arange(EP - E, dtype=jnp.int32) % (NR - N))
    srcp = jnp.concatenate([src, pad_src])
    dstp = jnp.concatenate([dst, jnp.zeros((EP - E,), jnp.int32)])
    src2d = srcp.reshape(32 * (EPT // 128), 128)
    cp = _edge_hist_sc(src2d, dstp, batch_i32, NR)
    c0 = cp[0, :N]
    c1 = cp[1, :N]

    thW = p['theta_W']
    thh = jnp.tile(thW[:H], (1, _NG))
    thl = jnp.tile(thW[H:H + laplacePE.shape[1]], (1, _NG))
    thr = jnp.tile(thW[H + laplacePE.shape[1]:], (1, _NG))
    row = lambda a: a.reshape(1, -1)
    full = lambda a: pl.BlockSpec(a.shape, lambda i: (0,) * a.ndim)
    wA = [p['enc_W1'], row(p['enc_b1']), row(p['enc_g']), row(p['enc_bn']),
          p['enc_W2'], row(p['enc_b2']), thh, thl, thr]
    h_full, axh, axl, axr, m_acc, cnt = pl.pallas_call(
        _enc_body,
        grid=(NB,),
        in_specs=[
            pl.BlockSpec((BN, D), lambda i: (i, 0)),
            pl.BlockSpec((BN, 1), lambda i: (i, 0)),
            pl.BlockSpec((BN, laplacePE.shape[1]), lambda i: (i, 0)),
            pl.BlockSpec((BN, rwse.shape[1]), lambda i: (i, 0)),
        ] + [full(a) for a in wA],
        out_specs=[
            pl.BlockSpec((BN, H), lambda i: (i, 0)),
            pl.BlockSpec((4 * _NG, H), lambda i: (0, 0)),
            pl.BlockSpec((4 * _NG, laplacePE.shape[1]), lambda i: (0, 0)),
            pl.BlockSpec((4 * _NG, rwse.shape[1]), lambda i: (0, 0)),
            pl.BlockSpec((_NG, H), lambda i: (0, 0)),
            pl.BlockSpec((_NG, 1), lambda i: (0, 0)),
        ],
        out_shape=[
            jax.ShapeDtypeStruct((N, H), _F32),
            jax.ShapeDtypeStruct((4 * _NG, H), _F32),
            jax.ShapeDtypeStruct((4 * _NG, laplacePE.shape[1]), _F32),
            jax.ShapeDtypeStruct((4 * _NG, rwse.shape[1]), _F32),
            jax.ShapeDtypeStruct((_NG, H), _F32),
            jax.ShapeDtypeStruct((_NG, 1), _F32),
        ],
    )(x, b2d, laplacePE, rwse, *wA)

    mc_acc, oc_acc = pl.pallas_call(
        _cred_body,
        grid=(NB,),
        in_specs=[
            pl.BlockSpec((BN, H), lambda i: (i, 0)),
            pl.BlockSpec((BN, 1), lambda i: (i, 0)),
            pl.BlockSpec((BN, _NG), lambda i: (i, 0)),
            pl.BlockSpec((BN, _NG), lambda i: (i, 0)),
        ],
        out_specs=[
            pl.BlockSpec((_NG, H), lambda i: (0, 0)),
            pl.BlockSpec((_NG, _NG), lambda i: (0, 0)),
        ],
        out_shape=[
            jax.ShapeDtypeStruct((_NG, H), _F32),
            jax.ShapeDtypeStruct((_NG, _NG), _F32),
        ],
    )(h_full, b2d, c0, c1)

    ax = jnp.concatenate([axh, axl, axr], axis=1)
    ax = ax.reshape(_NG, 4, dm).transpose(1, 0, 2)
    u0, u1, u2, u3 = ax[0], ax[1], ax[2], ax[3]

    wB = [p['in_proj_W'], p['conv_W'].T, row(p['conv_b']), p['x_proj_W'],
          p['dt_W'], row(p['dt_b']), p['A_log'], row(p['Dp']),
          p['out_proj_W'], row(p['lnm_g']), row(p['lnm_b']),
          p['merge_W'][:H], p['merge_W'][H:], row(p['merge_b']),
          p['gin2_W'], row(p['gin2_b']),
          p['dec_W1'], row(p['dec_b1']), row(p['dec_g']), row(p['dec_bn']),
          p['dec_W2'], row(p['dec_b2'])]
    out = pl.pallas_call(
        _mamba_body,
        out_shape=jax.ShapeDtypeStruct((_NG, OUT), _F32),
    )(u0, u1, u2, u3, m_acc, mc_acc, oc_acc, cnt, *wB)
    return out


# speedup vs baseline: 1.4977x; 1.4977x over previous
import functools

import jax
import jax.numpy as jnp
from jax import lax
from jax.experimental import pallas as pl
from jax.experimental.pallas import tpu as pltpu
from jax.experimental.pallas import tpu_sc as plsc

_NG = 16
_F32 = jnp.float32

_dot = functools.partial(
    lax.dot, precision=lax.Precision.DEFAULT, preferred_element_type=_F32)


def _dott(a, b):
    return lax.dot_general(
        a, b, (((0,), (0,)), ((), ())),
        precision=lax.Precision.DEFAULT, preferred_element_type=_F32)


def _ln(x, g, b):
    m = jnp.mean(x, axis=-1, keepdims=True)
    v = jnp.mean((x - m) ** 2, axis=-1, keepdims=True)
    return (x - m) * lax.rsqrt(v + 1e-5) * g + b


def _edge_hist_sc(src2d, dstp, batch, n_rows):
    N = batch.shape[0]
    EP = dstp.shape[0]
    NW = 32
    EPT = EP // NW
    NDMA = EPT // 128
    GROUPS = EPT // 16
    RPT = n_rows // 16
    mesh = plsc.VectorSubcoreMesh(core_axis_name="c", subcore_axis_name="s")

    @functools.partial(
        pl.kernel,
        mesh=mesh,
        compiler_params=pltpu.CompilerParams(
            needs_layout_passes=False, use_tc_tiling_on_sc=False),
        out_type=jax.ShapeDtypeStruct((2, n_rows, _NG), _F32),
        scratch_types=[
            pltpu.VMEM((NDMA, 128), jnp.int32),
            pltpu.VMEM((EPT,), jnp.int32),
            pltpu.VMEM((N,), jnp.int32),
            pltpu.VMEM((EPT, _NG), _F32),
            pltpu.VMEM_SHARED((n_rows, _NG), _F32),
            pltpu.SemaphoreType.DMA,
        ],
    )
    def k(src_hbm, dst_hbm, batch_hbm, out_hbm, srcv, dstv, bv, oh, csh, sem):
        c = lax.axis_index("c")
        s = lax.axis_index("s")
        tile = c * 16 + s
        pltpu.sync_copy(src_hbm.at[pl.ds(tile * NDMA, NDMA)], srcv)
        pltpu.sync_copy(dst_hbm.at[pl.ds(tile * EPT, EPT)], dstv)
        pltpu.sync_copy(batch_hbm, bv)

        zero16 = jnp.zeros((16,), _F32)

        def zero_body(r, _):
            oh[r, :] = zero16
            return 0

        lax.fori_loop(0, EPT, zero_body, 0)
        pltpu.sync_copy(oh.at[pl.ds(0, RPT)], csh.at[pl.ds(s * RPT, RPT)])
        plsc.subcore_barrier()

        ones16 = jnp.ones((16,), _F32)
        iota16 = lax.iota(jnp.int32, 16)

        def fill_body(g, _):
            dvec = dstv[pl.ds(g * 16, 16)]
            bg = plsc.load_gather(bv, [dvec])
            rows = g * 16 + iota16
            plsc.store_scatter(oh, [rows, bg], ones16)
            return 0

        lax.fori_loop(0, GROUPS, fill_body, 0)

        def fire_body(j, _):
            pltpu.async_copy(
                oh.at[pl.ds(j * 128, 128)], csh.at[srcv.at[j]], sem, add=True)
            return 0

        lax.fori_loop(0, NDMA, fire_body, 0)

        def drain_body(j, _):
            pltpu.make_async_copy(
                oh.at[pl.ds(j * 128, 128)], csh.at[srcv.at[j]], sem).wait()
            return 0

        lax.fori_loop(0, NDMA, drain_body, 0)
        plsc.subcore_barrier()

        pltpu.sync_copy(csh.at[pl.ds(s * RPT, RPT)], oh.at[pl.ds(0, RPT)])
        pltpu.sync_copy(oh.at[pl.ds(0, RPT)], out_hbm.at[c, pl.ds(s * RPT, RPT)])

    return k(src2d, dstp, batch)


def _enc_body(x_r, b_r, lpe_r, rw_r, W1, b1, g1, bn1, W2, b2,
              thh, thl, thr, h_out, axh, axl, axr, m_out, cnt_out):
    i = pl.program_id(0)
    t = _dot(x_r[...], W1[...]) + b1[...]
    t = jnp.maximum(_ln(t, g1[...], bn1[...]), 0.0)
    h = _dot(t, W2[...]) + b2[...]
    h_out[...] = h
    lpe = lpe_r[...]
    rw = rw_r[...]
    alpha64 = _dot(h, thh[...]) + _dot(lpe, thl[...]) + _dot(rw, thr[...])
    g64 = lax.broadcasted_iota(jnp.int32, (1, 4 * _NG), 1) // 4
    a2 = alpha64 * (b_r[...] == g64).astype(_F32)
    iota_g = lax.broadcasted_iota(jnp.int32, (1, _NG), 1)
    oh = (b_r[...] == iota_g).astype(_F32)

    @pl.when(i == 0)
    def _():
        axh[...] = jnp.zeros_like(axh)
        axl[...] = jnp.zeros_like(axl)
        axr[...] = jnp.zeros_like(axr)
        m_out[...] = jnp.zeros_like(m_out)
        cnt_out[...] = jnp.zeros_like(cnt_out)

    axh[...] += _dott(a2, h)
    axl[...] += _dott(a2, lpe)
    axr[...] += _dott(a2, rw)
    m_out[...] += _dott(oh, h)
    cnt_out[...] += _dott(oh, jnp.ones((h.shape[0], 1), _F32))


def _cred_body(h_r, b_r, c0_r, c1_r, mc_out, oc_out):
    i = pl.program_id(0)
    h = h_r[...]
    iota_g = lax.broadcasted_iota(jnp.int32, (1, _NG), 1)
    oh = (b_r[...] == iota_g).astype(_F32)
    cs = c0_r[...] + c1_r[...]

    @pl.when(i == 0)
    def _():
        mc_out[...] = jnp.zeros_like(mc_out)
        oc_out[...] = jnp.zeros_like(oc_out)

    mc_out[...] += _dott(cs, h)
    oc_out[...] += _dott(cs, oh)


def _mamba_body(u0, u1, u2, u3, m_r, mc_r, oc_r, cnt_r, inW, convWT, convb,
                xpW, dtW, dtb, Alog, Dp, outW, lng, lnb, mW1, mW2, mb, gW, gb,
                dW1, db1, dg, dbn, dW2, db2, out_r):
    us = [u0[...], u1[...], u2[...], u3[...]]
    di = convb.shape[-1]
    dr = dtW.shape[0]
    ds = Alog.shape[-1]
    xis = []
    z3 = None
    for t in range(4):
        xz = _dot(us[t], inW[...])
        xis.append(xz[:, :di])
        if t == 3:
            z3 = xz[:, di:]
    convs = []
    for t in range(4):
        acc = convb[...]
        for k in range(4):
            ti = t + k - 3
            if ti >= 0:
                acc = acc + xis[ti] * convWT[k, :][None, :]
        convs.append(acc * jax.nn.sigmoid(acc))
    dts, bms = [], []
    cm3 = None
    for t in range(4):
        dbc = _dot(convs[t], xpW[...])
        dts.append(jax.nn.softplus(_dot(dbc[:, :dr], dtW[...]) + dtb[...]))
        bms.append(dbc[:, dr:dr + ds])
        if t == 3:
            cm3 = dbc[:, dr + ds:dr + 2 * ds]
    A = -jnp.exp(Alog[...])
    h = jnp.zeros((us[0].shape[0], di, ds), _F32)
    for t in range(4):
        dA = jnp.exp(dts[t][:, :, None] * A[None, :, :])
        h = dA * h + (dts[t] * convs[t])[:, :, None] * bms[t][:, None, :]
    y3 = jnp.sum(h * cm3[:, None, :], axis=-1) + Dp[...] * convs[3]
    y3 = y3 * (z3 * jax.nn.sigmoid(z3))
    xm = _ln(_dot(y3, outW[...]), lng[...], lnb[...])
    q = _dot(xm, mW2[...]) + mb[...]

    cnt = cnt_r[...]
    iota_r = lax.broadcasted_iota(jnp.int32, (_NG, _NG), 0)
    iota_c = lax.broadcasted_iota(jnp.int32, (_NG, _NG), 1)
    o_tot = jnp.where(iota_r == iota_c, cnt, 0.0) + oc_r[...]
    s_tot = _dot(m_r[...] + mc_r[...], mW1[...]) + _dot(o_tot, q)
    pooled = _dot(s_tot, gW[...]) + cnt_r[...] * gb[...]
    t2 = _dot(pooled, dW1[...]) + db1[...]
    t2 = jnp.maximum(_ln(t2, dg[...], dbn[...]), 0.0)
    out_r[...] = _dot(t2, dW2[...]) + db2[...]


def kernel(x, edge_index, batch, laplacePE, rwse, params):
    p = params
    N, D = x.shape
    E = edge_index.shape[1]
    H = p['enc_W2'].shape[1]
    PE = laplacePE.shape[1] + rwse.shape[1]
    dm = H + PE
    OUT = p['dec_W2'].shape[1]
    BN = 2000
    NB = N // BN

    batch_i32 = batch.astype(jnp.int32)
    b2d = batch_i32.reshape(N, 1)

    EPT = 5120
    EP = 32 * EPT
    NR = ((N + 16 + 127) // 128) * 128
    src = edge_index[0].astype(jnp.int32)
    dst = edge_index[1].astype(jnp.int32)
    pad_src = N + (jnp.arange(EP - E, dtype=jnp.int32) % (NR - N))
    srcp = jnp.concatenate([src, pad_src])
    dstp = jnp.concatenate([dst, jnp.zeros((EP - E,), jnp.int32)])
    src2d = srcp.reshape(32 * (EPT // 128), 128)
    cp = _edge_hist_sc(src2d, dstp, batch_i32, NR)
    c0 = cp[0, :N]
    c1 = cp[1, :N]

    thW = p['theta_W']
    thh = jnp.tile(thW[:H], (1, _NG))
    thl = jnp.tile(thW[H:H + laplacePE.shape[1]], (1, _NG))
    thr = jnp.tile(thW[H + laplacePE.shape[1]:], (1, _NG))
    row = lambda a: a.reshape(1, -1)
    full = lambda a: pl.BlockSpec(a.shape, lambda i: (0,) * a.ndim)
    wA = [p['enc_W1'], row(p['enc_b1']), row(p['enc_g']), row(p['enc_bn']),
          p['enc_W2'], row(p['enc_b2']), thh, thl, thr]
    h_full, axh, axl, axr, m_acc, cnt = pl.pallas_call(
        _enc_body,
        grid=(NB,),
        in_specs=[
            pl.BlockSpec((BN, D), lambda i: (i, 0)),
            pl.BlockSpec((BN, 1), lambda i: (i, 0)),
            pl.BlockSpec((BN, laplacePE.shape[1]), lambda i: (i, 0)),
            pl.BlockSpec((BN, rwse.shape[1]), lambda i: (i, 0)),
        ] + [full(a) for a in wA],
        out_specs=[
            pl.BlockSpec((BN, H), lambda i: (i, 0)),
            pl.BlockSpec((4 * _NG, H), lambda i: (0, 0)),
            pl.BlockSpec((4 * _NG, laplacePE.shape[1]), lambda i: (0, 0)),
            pl.BlockSpec((4 * _NG, rwse.shape[1]), lambda i: (0, 0)),
            pl.BlockSpec((_NG, H), lambda i: (0, 0)),
            pl.BlockSpec((_NG, 1), lambda i: (0, 0)),
        ],
        out_shape=[
            jax.ShapeDtypeStruct((N, H), _F32),
            jax.ShapeDtypeStruct((4 * _NG, H), _F32),
            jax.ShapeDtypeStruct((4 * _NG, laplacePE.shape[1]), _F32),
            jax.ShapeDtypeStruct((4 * _NG, rwse.shape[1]), _F32),
            jax.ShapeDtypeStruct((_NG, H), _F32),
            jax.ShapeDtypeStruct((_NG, 1), _F32),
        ],
    )(x, b2d, laplacePE, rwse, *wA)

    mc_acc, oc_acc = pl.pallas_call(
        _cred_body,
        grid=(NB,),
        in_specs=[
            pl.BlockSpec((BN, H), lambda i: (i, 0)),
            pl.BlockSpec((BN, 1), lambda i: (i, 0)),
            pl.BlockSpec((BN, _NG), lambda i: (i, 0)),
            pl.BlockSpec((BN, _NG), lambda i: (i, 0)),
        ],
        out_specs=[
            pl.BlockSpec((_NG, H), lambda i: (0, 0)),
            pl.BlockSpec((_NG, _NG), lambda i: (0, 0)),
        ],
        out_shape=[
            jax.ShapeDtypeStruct((_NG, H), _F32),
            jax.ShapeDtypeStruct((_NG, _NG), _F32),
        ],
    )(h_full, b2d, c0, c1)

    ax = jnp.concatenate([axh, axl, axr], axis=1)
    ax = ax.reshape(_NG, 4, dm).transpose(1, 0, 2)
    u0, u1, u2, u3 = ax[0], ax[1], ax[2], ax[3]

    wB = [p['in_proj_W'], p['conv_W'].T, row(p['conv_b']), p['x_proj_W'],
          p['dt_W'], row(p['dt_b']), p['A_log'], row(p['Dp']),
          p['out_proj_W'], row(p['lnm_g']), row(p['lnm_b']),
          p['merge_W'][:H], p['merge_W'][H:], row(p['merge_b']),
          p['gin2_W'], row(p['gin2_b']),
          p['dec_W1'], row(p['dec_b1']), row(p['dec_g']), row(p['dec_bn']),
          p['dec_W2'], row(p['dec_b2'])]
    out = pl.pallas_call(
        _mamba_body,
        out_shape=jax.ShapeDtypeStruct((_NG, OUT), _F32),
    )(u0, u1, u2, u3, m_acc, mc_acc, oc_acc, cnt, *wB)
    return out


# speedup vs baseline: 1.5771x; 1.0531x over previous
import functools

import jax
import jax.numpy as jnp
from jax import lax
from jax.experimental import pallas as pl
from jax.experimental.pallas import tpu as pltpu
from jax.experimental.pallas import tpu_sc as plsc

_NG = 16
_F32 = jnp.float32

_dot = functools.partial(
    lax.dot, precision=lax.Precision.DEFAULT, preferred_element_type=_F32)


def _dott(a, b):
    return lax.dot_general(
        a, b, (((0,), (0,)), ((), ())),
        precision=lax.Precision.DEFAULT, preferred_element_type=_F32)


def _ln(x, g, b):
    m = jnp.mean(x, axis=-1, keepdims=True)
    v = jnp.mean((x - m) ** 2, axis=-1, keepdims=True)
    return (x - m) * lax.rsqrt(v + 1e-5) * g + b


def _edge_hist_sc(src2d, dstp, batch, n_rows):
    N = batch.shape[0]
    EP = dstp.shape[0]
    NW = 32
    EPT = EP // NW
    NDMA = EPT // 128
    GROUPS = EPT // 16
    RPT = n_rows // 16
    mesh = plsc.VectorSubcoreMesh(core_axis_name="c", subcore_axis_name="s")

    @functools.partial(
        pl.kernel,
        mesh=mesh,
        compiler_params=pltpu.CompilerParams(
            needs_layout_passes=False, use_tc_tiling_on_sc=False),
        out_type=jax.ShapeDtypeStruct((2, n_rows, _NG), _F32),
        scratch_types=[
            pltpu.VMEM((NDMA, 128), jnp.int32),
            pltpu.VMEM((EPT,), jnp.int32),
            pltpu.VMEM((N,), jnp.int32),
            pltpu.VMEM((EPT, _NG), _F32),
            pltpu.VMEM_SHARED((n_rows, _NG), _F32),
            pltpu.SemaphoreType.DMA,
        ],
    )
    def k(src_hbm, dst_hbm, batch_hbm, out_hbm, srcv, dstv, bv, oh, csh, sem):
        c = lax.axis_index("c")
        s = lax.axis_index("s")
        tile = c * 16 + s
        pltpu.sync_copy(src_hbm.at[pl.ds(tile * NDMA, NDMA)], srcv)
        pltpu.sync_copy(dst_hbm.at[pl.ds(tile * EPT, EPT)], dstv)
        pltpu.sync_copy(batch_hbm, bv)

        zero16 = jnp.zeros((16,), _F32)

        def zero_body(r, _):
            oh[r, :] = zero16
            return 0

        lax.fori_loop(0, EPT, zero_body, 0)
        pltpu.sync_copy(oh.at[pl.ds(0, RPT)], csh.at[pl.ds(s * RPT, RPT)])
        plsc.subcore_barrier()

        ones16 = jnp.ones((16,), _F32)
        iota16 = lax.iota(jnp.int32, 16)

        def fill_body(g, _):
            dvec = dstv[pl.ds(g * 16, 16)]
            bg = plsc.load_gather(bv, [dvec])
            rows = g * 16 + iota16
            plsc.store_scatter(oh, [rows, bg], ones16)
            return 0

        lax.fori_loop(0, GROUPS, fill_body, 0)

        def fire_body(j, _):
            pltpu.async_copy(
                oh.at[pl.ds(j * 128, 128)], csh.at[srcv.at[j]], sem, add=True)
            return 0

        lax.fori_loop(0, NDMA, fire_body, 0)

        def drain_body(j, _):
            pltpu.make_async_copy(
                oh.at[pl.ds(j * 128, 128)], csh.at[srcv.at[j]], sem).wait()
            return 0

        lax.fori_loop(0, NDMA, drain_body, 0)
        plsc.subcore_barrier()

        pltpu.sync_copy(csh.at[pl.ds(s * RPT, RPT)], oh.at[pl.ds(0, RPT)])
        pltpu.sync_copy(oh.at[pl.ds(0, RPT)], out_hbm.at[c, pl.ds(s * RPT, RPT)])

    return k(src2d, dstp, batch)


def _enc_body(x_r, b_r, lpe_r, rw_r, W1, b1, g1, bn1, W2, b2,
              thh, thl, thr, h_out, axh, axl, axr, m_out, cnt_out):
    i = pl.program_id(0)
    t = _dot(x_r[...], W1[...]) + b1[...]
    t = jnp.maximum(_ln(t, g1[...], bn1[...]), 0.0)
    h = _dot(t, W2[...]) + b2[...]
    h_out[...] = h
    lpe = lpe_r[...]
    rw = rw_r[...]
    alpha64 = _dot(h, thh[...]) + _dot(lpe, thl[...]) + _dot(rw, thr[...])
    g64 = lax.broadcasted_iota(jnp.int32, (1, 4 * _NG), 1) // 4
    a2 = alpha64 * (b_r[...] == g64).astype(_F32)
    iota_g = lax.broadcasted_iota(jnp.int32, (1, _NG), 1)
    oh = (b_r[...] == iota_g).astype(_F32)

    @pl.when(i == 0)
    def _():
        axh[...] = jnp.zeros_like(axh)
        axl[...] = jnp.zeros_like(axl)
        axr[...] = jnp.zeros_like(axr)
        m_out[...] = jnp.zeros_like(m_out)
        cnt_out[...] = jnp.zeros_like(cnt_out)

    axh[...] += _dott(a2, h)
    axl[...] += _dott(a2, lpe)
    axr[...] += _dott(a2, rw)
    m_out[...] += _dott(oh, h)
    cnt_out[...] += _dott(oh, jnp.ones((h.shape[0], 1), _F32))


def _cred_body(nb, h_r, b_r, c0_r, c1_r, u0, u1, u2, u3, m_r, cnt_r, inW,
               convWT, convb, xpW, dtW, dtb, Alog, Dp, outW, lng, lnb, mW1,
               mW2, mb, gW, gb, dW1, db1, dg, dbn, dW2, db2, out_r,
               mc_s, oc_s):
    i = pl.program_id(0)

    @pl.when(i == 0)
    def _():
        mc_s[...] = jnp.zeros_like(mc_s)
        oc_s[...] = jnp.zeros_like(oc_s)

    @pl.when(i < nb)
    def _():
        h = h_r[...]
        iota_g = lax.broadcasted_iota(jnp.int32, (1, _NG), 1)
        oh = (b_r[...] == iota_g).astype(_F32)
        cs = c0_r[...] + c1_r[...]
        mc_s[...] += _dott(cs, h)
        oc_s[...] += _dott(cs, oh)

    @pl.when(i == nb)
    def _():
        _mamba_final(u0, u1, u2, u3, m_r, mc_s, oc_s, cnt_r, inW, convWT,
                     convb, xpW, dtW, dtb, Alog, Dp, outW, lng, lnb, mW1,
                     mW2, mb, gW, gb, dW1, db1, dg, dbn, dW2, db2, out_r)


def _mamba_final(u0, u1, u2, u3, m_r, mc_r, oc_r, cnt_r, inW, convWT, convb,
                 xpW, dtW, dtb, Alog, Dp, outW, lng, lnb, mW1, mW2, mb, gW, gb,
                 dW1, db1, dg, dbn, dW2, db2, out_r):
    us = [u0[...], u1[...], u2[...], u3[...]]
    di = convb.shape[-1]
    dr = dtW.shape[0]
    ds = Alog.shape[-1]
    xis = []
    z3 = None
    for t in range(4):
        xz = _dot(us[t], inW[...])
        xis.append(xz[:, :di])
        if t == 3:
            z3 = xz[:, di:]
    convs = []
    for t in range(4):
        acc = convb[...]
        for k in range(4):
            ti = t + k - 3
            if ti >= 0:
                acc = acc + xis[ti] * convWT[k, :][None, :]
        convs.append(acc * jax.nn.sigmoid(acc))
    dts, bms = [], []
    cm3 = None
    for t in range(4):
        dbc = _dot(convs[t], xpW[...])
        dts.append(jax.nn.softplus(_dot(dbc[:, :dr], dtW[...]) + dtb[...]))
        bms.append(dbc[:, dr:dr + ds])
        if t == 3:
            cm3 = dbc[:, dr + ds:dr + 2 * ds]
    A = -jnp.exp(Alog[...])
    NGn = us[0].shape[0]
    edges = list(range(0, di, 256)) + [di]
    y3parts = []
    for lo, hi in zip(edges[:-1], edges[1:]):
        sl = slice(lo, hi)
        Ac = A[sl, :]
        h = jnp.zeros((NGn, hi - lo, ds), _F32)
        for t in range(4):
            dA = jnp.exp(dts[t][:, sl, None] * Ac[None, :, :])
            h = dA * h + (dts[t][:, sl] * convs[t][:, sl])[:, :, None] \
                * bms[t][:, None, :]
        y3parts.append(jnp.sum(h * cm3[:, None, :], axis=-1))
    y3 = jnp.concatenate(y3parts, axis=1) + Dp[...] * convs[3]
    y3 = y3 * (z3 * jax.nn.sigmoid(z3))
    xm = _ln(_dot(y3, outW[...]), lng[...], lnb[...])
    q = _dot(xm, mW2[...]) + mb[...]

    cnt = cnt_r[...]
    iota_r = lax.broadcasted_iota(jnp.int32, (_NG, _NG), 0)
    iota_c = lax.broadcasted_iota(jnp.int32, (_NG, _NG), 1)
    o_tot = jnp.where(iota_r == iota_c, cnt, 0.0) + oc_r[...]
    s_tot = _dot(m_r[...] + mc_r[...], mW1[...]) + _dot(o_tot, q)
    pooled = _dot(s_tot, gW[...]) + cnt_r[...] * gb[...]
    t2 = _dot(pooled, dW1[...]) + db1[...]
    t2 = jnp.maximum(_ln(t2, dg[...], dbn[...]), 0.0)
    out_r[...] = _dot(t2, dW2[...]) + db2[...]


def kernel(x, edge_index, batch, laplacePE, rwse, params):
    p = params
    N, D = x.shape
    E = edge_index.shape[1]
    H = p['enc_W2'].shape[1]
    PE = laplacePE.shape[1] + rwse.shape[1]
    dm = H + PE
    OUT = p['dec_W2'].shape[1]
    BN = 2000
    NB = N // BN

    batch_i32 = batch.astype(jnp.int32)
    b2d = batch_i32.reshape(N, 1)

    EPT = 5120
    EP = 32 * EPT
    NR = ((N + 16 + 127) // 128) * 128
    src = edge_index[0].astype(jnp.int32)
    dst = edge_index[1].astype(jnp.int32)
    pad_src = N + (jnp.arange(EP - E, dtype=jnp.int32) % (NR - N))
    srcp = jnp.concatenate([src, pad_src])
    dstp = jnp.concatenate([dst, jnp.zeros((EP - E,), jnp.int32)])
    src2d = srcp.reshape(32 * (EPT // 128), 128)
    cp = _edge_hist_sc(src2d, dstp, batch_i32, NR)
    c0 = cp[0, :N]
    c1 = cp[1, :N]

    thW = p['theta_W']
    thh = jnp.tile(thW[:H], (1, _NG))
    thl = jnp.tile(thW[H:H + laplacePE.shape[1]], (1, _NG))
    thr = jnp.tile(thW[H + laplacePE.shape[1]:], (1, _NG))
    row = lambda a: a.reshape(1, -1)
    full = lambda a: pl.BlockSpec(a.shape, lambda i: (0,) * a.ndim)
    wA = [p['enc_W1'], row(p['enc_b1']), row(p['enc_g']), row(p['enc_bn']),
          p['enc_W2'], row(p['enc_b2']), thh, thl, thr]
    h_full, axh, axl, axr, m_acc, cnt = pl.pallas_call(
        _enc_body,
        grid=(NB,),
        in_specs=[
            pl.BlockSpec((BN, D), lambda i: (i, 0)),
            pl.BlockSpec((BN, 1), lambda i: (i, 0)),
            pl.BlockSpec((BN, laplacePE.shape[1]), lambda i: (i, 0)),
            pl.BlockSpec((BN, rwse.shape[1]), lambda i: (i, 0)),
        ] + [full(a) for a in wA],
        out_specs=[
            pl.BlockSpec((BN, H), lambda i: (i, 0)),
            pl.BlockSpec((4 * _NG, H), lambda i: (0, 0)),
            pl.BlockSpec((4 * _NG, laplacePE.shape[1]), lambda i: (0, 0)),
            pl.BlockSpec((4 * _NG, rwse.shape[1]), lambda i: (0, 0)),
            pl.BlockSpec((_NG, H), lambda i: (0, 0)),
            pl.BlockSpec((_NG, 1), lambda i: (0, 0)),
        ],
        out_shape=[
            jax.ShapeDtypeStruct((N, H), _F32),
            jax.ShapeDtypeStruct((4 * _NG, H), _F32),
            jax.ShapeDtypeStruct((4 * _NG, laplacePE.shape[1]), _F32),
            jax.ShapeDtypeStruct((4 * _NG, rwse.shape[1]), _F32),
            jax.ShapeDtypeStruct((_NG, H), _F32),
            jax.ShapeDtypeStruct((_NG, 1), _F32),
        ],
    )(x, b2d, laplacePE, rwse, *wA)

    ax = jnp.concatenate([axh, axl, axr], axis=1)
    ax = ax.reshape(_NG, 4, dm).transpose(1, 0, 2)
    u0, u1, u2, u3 = ax[0], ax[1], ax[2], ax[3]

    wB = [p['in_proj_W'], p['conv_W'].T, row(p['conv_b']), p['x_proj_W'],
          p['dt_W'], row(p['dt_b']), p['A_log'], row(p['Dp']),
          p['out_proj_W'], row(p['lnm_g']), row(p['lnm_b']),
          p['merge_W'][:H], p['merge_W'][H:], row(p['merge_b']),
          p['gin2_W'], row(p['gin2_b']),
          p['dec_W1'], row(p['dec_b1']), row(p['dec_g']), row(p['dec_bn']),
          p['dec_W2'], row(p['dec_b2'])]
    clamp = lambda i: (jnp.minimum(i, NB - 1), 0)
    out = pl.pallas_call(
        functools.partial(_cred_body, NB),
        grid=(NB + 1,),
        in_specs=[
            pl.BlockSpec((BN, H), clamp),
            pl.BlockSpec((BN, 1), clamp),
            pl.BlockSpec((BN, _NG), clamp),
            pl.BlockSpec((BN, _NG), clamp),
        ] + [full(a) for a in [u0, u1, u2, u3, m_acc, cnt] + wB],
        out_specs=pl.BlockSpec((_NG, OUT), lambda i: (0, 0)),
        out_shape=jax.ShapeDtypeStruct((_NG, OUT), _F32),
        scratch_shapes=[
            pltpu.VMEM((_NG, H), _F32),
            pltpu.VMEM((_NG, _NG), _F32),
        ],
    )(h_full, b2d, c0, c1, u0, u1, u2, u3, m_acc, cnt, *wB)
    return out
